# baseline (speedup 1.0000x reference)
import jax
import jax.numpy as jnp
from jax.experimental import pallas as pl

_B, _N, _FD = 8, 4096, 128
_M, _K = 1024, 32
_GH, _FH = 256, 256

_ROWS = 512


def _fps_body(x_ref, y_ref, z_ref, cx_ref, cy_ref, cz_ref):
    X = x_ref[...]
    Y = y_ref[...]
    Z = z_ref[...]
    lane = jax.lax.broadcasted_iota(jnp.int32, (_B, _N), 1)
    big = jnp.int32(_N)

    mlane = jax.lax.broadcasted_iota(jnp.int32, (_B, _M), 1)

    first = lane == 0
    cx0 = jnp.sum(jnp.where(first, X, 0.0), axis=1, keepdims=True)
    cy0 = jnp.sum(jnp.where(first, Y, 0.0), axis=1, keepdims=True)
    cz0 = jnp.sum(jnp.where(first, Z, 0.0), axis=1, keepdims=True)
    zerosm = jnp.zeros((_B, _M), jnp.float32)
    cxs0 = jnp.where(mlane == 0, cx0, zerosm)
    cys0 = jnp.where(mlane == 0, cy0, zerosm)
    czs0 = jnp.where(mlane == 0, cz0, zerosm)

    dists0 = jnp.full((_B, _N), 1e10, dtype=jnp.float32)

    def body(i, state):
        dists, lx, ly, lz, cxs, cys, czs = state
        d = (X - lx) ** 2 + (Y - ly) ** 2 + (Z - lz) ** 2
        dists = jnp.minimum(dists, d)
        m = jnp.max(dists, axis=1, keepdims=True)
        jstar = jnp.min(jnp.where(dists == m, lane, big), axis=1, keepdims=True)
        sel = lane == jstar
        nx = jnp.sum(jnp.where(sel, X, 0.0), axis=1, keepdims=True)
        ny = jnp.sum(jnp.where(sel, Y, 0.0), axis=1, keepdims=True)
        nz = jnp.sum(jnp.where(sel, Z, 0.0), axis=1, keepdims=True)
        here = mlane == i
        cxs = jnp.where(here, nx, cxs)
        cys = jnp.where(here, ny, cys)
        czs = jnp.where(here, nz, czs)
        return dists, nx, ny, nz, cxs, cys, czs

    _, _, _, _, cxs, cys, czs = jax.lax.fori_loop(
        1, _M, body, (dists0, cx0, cy0, cz0, cxs0, cys0, czs0))
    cx_ref[...] = cxs
    cy_ref[...] = cys
    cz_ref[...] = czs


def _fps_centroids(xyz):
    x = xyz[:, :, 0]
    y = xyz[:, :, 1]
    z = xyz[:, :, 2]
    shp = jax.ShapeDtypeStruct((_B, _M), jnp.float32)
    cx, cy, cz = pl.pallas_call(
        _fps_body,
        out_shape=(shp, shp, shp),
    )(x, y, z)
    return jnp.stack([cx, cy, cz], axis=-1)


def _mlp_block(dx_ref, dy_ref, dz_ref, xf_ref, w1x_ref, w1y_ref, w1z_ref,
               w1e2_ref, b1g_ref, w2gt_ref, w2ge_ref, w1ft_ref, b1f_ref,
               w2ft_ref, b2f_ref, out_ref):
    dx = dx_ref[...]
    dy = dy_ref[...]
    dz = dz_ref[...]
    e2 = -0.5 * (dx * dx + dy * dy + dz * dz)
    h = (jnp.dot(dx, w1x_ref[...], preferred_element_type=jnp.float32)
         + jnp.dot(dy, w1y_ref[...], preferred_element_type=jnp.float32)
         + jnp.dot(dz, w1z_ref[...], preferred_element_type=jnp.float32)
         + jnp.dot(e2, w1e2_ref[...], preferred_element_type=jnp.float32)
         + b1g_ref[...])
    e2h = -0.5 * jnp.sum(h * h, axis=1, keepdims=True)
    xgeo = (jnp.dot(h, w2gt_ref[...], preferred_element_type=jnp.float32)
            - w2ge_ref[0, :][None, :] + e2h * w2ge_ref[1, :][None, :])
    hf = jnp.maximum(
        jnp.dot(xf_ref[...], w1ft_ref[...], preferred_element_type=jnp.float32)
        + b1f_ref[...], 0.0)
    xfeat = jnp.dot(hf, w2ft_ref[...], preferred_element_type=jnp.float32) + b2f_ref[...]
    out_ref[:, :_GH] = xgeo
    out_ref[:, _GH:] = xfeat


def _mlps(dx, dy, dz, xf, W1g, W2g, W1f, b1f, W2f, b2f):
    nrows = dx.shape[0]
    grid = nrows // _ROWS
    w1gt = W1g.T
    w1x = w1gt[0::5]
    w1y = w1gt[1::5]
    w1z = w1gt[2::5]
    w1e2 = w1gt[4::5]
    b1g = -jnp.sum(w1gt[3::5], axis=0, keepdims=True)
    w2gt = W2g[:, :_GH].T
    w2ge = W2g[:, _GH:].T
    kspec = pl.BlockSpec((_K, _GH), lambda i: (0, 0))
    bspec = pl.BlockSpec((1, _FH), lambda i: (0, 0))
    rkspec = pl.BlockSpec((_ROWS, _K), lambda i: (i, 0))
    out = pl.pallas_call(
        _mlp_block,
        grid=(grid,),
        in_specs=[
            rkspec, rkspec, rkspec,
            pl.BlockSpec((_ROWS, _K * _FD), lambda i: (i, 0)),
            kspec, kspec, kspec, kspec,
            bspec,
            pl.BlockSpec((_GH, _GH), lambda i: (0, 0)),
            pl.BlockSpec((2, _GH), lambda i: (0, 0)),
            pl.BlockSpec((_K * _FD, _FH), lambda i: (0, 0)),
            bspec,
            pl.BlockSpec((_FH, _FH), lambda i: (0, 0)),
            bspec,
        ],
        out_specs=pl.BlockSpec((_ROWS, _GH + _FH), lambda i: (i, 0)),
        out_shape=jax.ShapeDtypeStruct((nrows, _GH + _FH), jnp.float32),
    )(dx, dy, dz, xf, w1x, w1y, w1z, w1e2, b1g, w2gt, w2ge, W1f.T,
      b1f.reshape(1, _FH), W2f.T, b2f.reshape(1, _FH))
    return out


def kernel(xyz, features, W1g, W2g, W1f, b1f, W2f, b2f):
    b, n, _ = xyz.shape
    centroids = _fps_centroids(xyz)
    d2 = (jnp.sum(centroids ** 2, axis=-1)[:, :, None]
          + jnp.sum(xyz ** 2, axis=-1)[:, None, :]
          - 2.0 * jnp.einsum('bmd,bnd->bmn', centroids, xyz))
    _, group_idx = jax.lax.top_k(-d2, _K)
    gi_flat = group_idx.reshape(b, _M * _K)

    def _coord(c):
        g = jnp.take_along_axis(xyz[:, :, c], gi_flat, axis=1).reshape(b, _M, _K)
        return (g - centroids[:, :, c:c + 1]).reshape(b * _M, _K)

    dx = _coord(0)
    dy = _coord(1)
    dz = _coord(2)
    grouped_feat = jax.vmap(lambda f, gi: f[gi])(features, group_idx)
    xf = grouped_feat.reshape(b * _M, _K * _FD)
    out = _mlps(dx, dy, dz, xf, W1g, W2g, W1f, b1f, W2f, b2f)
    return out.reshape(b, _M, _GH + _FH)


# speedup vs baseline: 2.7805x; 2.7805x over previous
import jax
import jax.numpy as jnp
from jax import lax
from jax.experimental import pallas as pl
from jax.experimental.pallas import tpu as pltpu
from jax.experimental.pallas import tpu_sc as plsc

_B, _N, _FD = 8, 4096, 128
_M, _K = 1024, 32
_GH, _FH = 256, 256

_ROWS = 512


def _fps_body(x_ref, y_ref, z_ref, cx_ref, cy_ref, cz_ref):
    X = x_ref[...]
    Y = y_ref[...]
    Z = z_ref[...]
    lane = jax.lax.broadcasted_iota(jnp.int32, (_B, _N), 1)
    big = jnp.int32(_N)

    mlane = jax.lax.broadcasted_iota(jnp.int32, (_B, _M), 1)

    first = lane == 0
    cx0 = jnp.sum(jnp.where(first, X, 0.0), axis=1, keepdims=True)
    cy0 = jnp.sum(jnp.where(first, Y, 0.0), axis=1, keepdims=True)
    cz0 = jnp.sum(jnp.where(first, Z, 0.0), axis=1, keepdims=True)
    zerosm = jnp.zeros((_B, _M), jnp.float32)
    cxs0 = jnp.where(mlane == 0, cx0, zerosm)
    cys0 = jnp.where(mlane == 0, cy0, zerosm)
    czs0 = jnp.where(mlane == 0, cz0, zerosm)

    dists0 = jnp.full((_B, _N), 1e10, dtype=jnp.float32)

    def body(i, state):
        dists, lx, ly, lz, cxs, cys, czs = state
        d = (X - lx) ** 2 + (Y - ly) ** 2 + (Z - lz) ** 2
        dists = jnp.minimum(dists, d)
        m = jnp.max(dists, axis=1, keepdims=True)
        jstar = jnp.min(jnp.where(dists == m, lane, big), axis=1, keepdims=True)
        sel = lane == jstar
        nx = jnp.sum(jnp.where(sel, X, 0.0), axis=1, keepdims=True)
        ny = jnp.sum(jnp.where(sel, Y, 0.0), axis=1, keepdims=True)
        nz = jnp.sum(jnp.where(sel, Z, 0.0), axis=1, keepdims=True)
        here = mlane == i
        cxs = jnp.where(here, nx, cxs)
        cys = jnp.where(here, ny, cys)
        czs = jnp.where(here, nz, czs)
        return dists, nx, ny, nz, cxs, cys, czs

    _, _, _, _, cxs, cys, czs = jax.lax.fori_loop(
        1, _M, body, (dists0, cx0, cy0, cz0, cxs0, cys0, czs0))
    cx_ref[...] = cxs
    cy_ref[...] = cys
    cz_ref[...] = czs


def _fps_centroids(xyz):
    x = xyz[:, :, 0]
    y = xyz[:, :, 1]
    z = xyz[:, :, 2]
    shp = jax.ShapeDtypeStruct((_B, _M), jnp.float32)
    cx, cy, cz = pl.pallas_call(
        _fps_body,
        out_shape=(shp, shp, shp),
    )(x, y, z)
    return jnp.stack([cx, cy, cz], axis=-1)


_NW = 32
_IPW = (_B * _M * _K) // _NW
_FCH = 256
_CCH = 2048


def _gather_body(feat_hbm, xp_hbm, yp_hbm, zp_hbm, idx_hbm,
                 xf_out, gx_out, gy_out, gz_out,
                 idx_v, idxf_v, gx_v, gy_v, gz_v, rows_v, sem):
    wid = lax.axis_index("s") * 2 + lax.axis_index("c")
    base = wid * _IPW

    def coord_chunk(c, carry):
        off = base + c * _CCH
        pltpu.sync_copy(idx_hbm.at[pl.ds(off, _CCH)], idx_v)
        pltpu.async_copy(xp_hbm.at[idx_v], gx_v, sem).wait()
        pltpu.async_copy(yp_hbm.at[idx_v], gy_v, sem).wait()
        pltpu.async_copy(zp_hbm.at[idx_v], gz_v, sem).wait()
        pltpu.sync_copy(gx_v, gx_out.at[pl.ds(off, _CCH)])
        pltpu.sync_copy(gy_v, gy_out.at[pl.ds(off, _CCH)])
        pltpu.sync_copy(gz_v, gz_out.at[pl.ds(off, _CCH)])
        return carry

    lax.fori_loop(0, _IPW // _CCH, coord_chunk, 0)

    def feat_chunk(c, carry):
        off = base + c * _FCH
        pltpu.sync_copy(idx_hbm.at[pl.ds(off, _FCH)], idxf_v)
        pltpu.async_copy(feat_hbm.at[idxf_v], rows_v, sem).wait()
        pltpu.sync_copy(rows_v, xf_out.at[pl.ds(off, _FCH)])
        return carry

    lax.fori_loop(0, _IPW // _FCH, feat_chunk, 0)


def _sc_gather(features_flat, xyz, idx_flat):
    mesh = plsc.VectorSubcoreMesh(core_axis_name="c", subcore_axis_name="s")
    nrows = _B * _M * _K
    k = pl.kernel(
        _gather_body,
        mesh=mesh,
        out_type=(
            jax.ShapeDtypeStruct((nrows, _FD), jnp.float32),
            jax.ShapeDtypeStruct((nrows,), jnp.float32),
            jax.ShapeDtypeStruct((nrows,), jnp.float32),
            jax.ShapeDtypeStruct((nrows,), jnp.float32),
        ),
        scratch_types=[
            pltpu.VMEM((_CCH,), jnp.int32),
            pltpu.VMEM((_FCH,), jnp.int32),
            pltpu.VMEM((_CCH,), jnp.float32),
            pltpu.VMEM((_CCH,), jnp.float32),
            pltpu.VMEM((_CCH,), jnp.float32),
            pltpu.VMEM((_FCH, _FD), jnp.float32),
            pltpu.SemaphoreType.DMA,
        ],
    )
    xflat = xyz[:, :, 0].reshape(_B * _N)
    yflat = xyz[:, :, 1].reshape(_B * _N)
    zflat = xyz[:, :, 2].reshape(_B * _N)
    return k(features_flat, xflat, yflat, zflat, idx_flat)


def _mlp_block(dx_ref, dy_ref, dz_ref, xf_ref, w1x_ref, w1y_ref, w1z_ref,
               w1e2_ref, b1g_ref, w2gt_ref, w2ge_ref, w1ft_ref, b1f_ref,
               w2ft_ref, b2f_ref, out_ref):
    dx = dx_ref[...]
    dy = dy_ref[...]
    dz = dz_ref[...]
    e2 = -0.5 * (dx * dx + dy * dy + dz * dz)
    h = (jnp.dot(dx, w1x_ref[...], preferred_element_type=jnp.float32)
         + jnp.dot(dy, w1y_ref[...], preferred_element_type=jnp.float32)
         + jnp.dot(dz, w1z_ref[...], preferred_element_type=jnp.float32)
         + jnp.dot(e2, w1e2_ref[...], preferred_element_type=jnp.float32)
         + b1g_ref[...])
    e2h = -0.5 * jnp.sum(h * h, axis=1, keepdims=True)
    xgeo = (jnp.dot(h, w2gt_ref[...], preferred_element_type=jnp.float32)
            - w2ge_ref[0, :][None, :] + e2h * w2ge_ref[1, :][None, :])
    hf = jnp.maximum(
        jnp.dot(xf_ref[...], w1ft_ref[...], preferred_element_type=jnp.float32)
        + b1f_ref[...], 0.0)
    xfeat = jnp.dot(hf, w2ft_ref[...], preferred_element_type=jnp.float32) + b2f_ref[...]
    out_ref[:, :_GH] = xgeo
    out_ref[:, _GH:] = xfeat


def _mlps(dx, dy, dz, xf, W1g, W2g, W1f, b1f, W2f, b2f):
    nrows = dx.shape[0]
    grid = nrows // _ROWS
    w1gt = W1g.T
    w1x = w1gt[0::5]
    w1y = w1gt[1::5]
    w1z = w1gt[2::5]
    w1e2 = w1gt[4::5]
    b1g = -jnp.sum(w1gt[3::5], axis=0, keepdims=True)
    w2gt = W2g[:, :_GH].T
    w2ge = W2g[:, _GH:].T
    kspec = pl.BlockSpec((_K, _GH), lambda i: (0, 0))
    bspec = pl.BlockSpec((1, _FH), lambda i: (0, 0))
    rkspec = pl.BlockSpec((_ROWS, _K), lambda i: (i, 0))
    out = pl.pallas_call(
        _mlp_block,
        grid=(grid,),
        in_specs=[
            rkspec, rkspec, rkspec,
            pl.BlockSpec((_ROWS, _K * _FD), lambda i: (i, 0)),
            kspec, kspec, kspec, kspec,
            bspec,
            pl.BlockSpec((_GH, _GH), lambda i: (0, 0)),
            pl.BlockSpec((2, _GH), lambda i: (0, 0)),
            pl.BlockSpec((_K * _FD, _FH), lambda i: (0, 0)),
            bspec,
            pl.BlockSpec((_FH, _FH), lambda i: (0, 0)),
            bspec,
        ],
        out_specs=pl.BlockSpec((_ROWS, _GH + _FH), lambda i: (i, 0)),
        out_shape=jax.ShapeDtypeStruct((nrows, _GH + _FH), jnp.float32),
    )(dx, dy, dz, xf, w1x, w1y, w1z, w1e2, b1g, w2gt, w2ge, W1f.T,
      b1f.reshape(1, _FH), W2f.T, b2f.reshape(1, _FH))
    return out


def kernel(xyz, features, W1g, W2g, W1f, b1f, W2f, b2f):
    b, n, _ = xyz.shape
    centroids = _fps_centroids(xyz)
    d2 = (jnp.sum(centroids ** 2, axis=-1)[:, :, None]
          + jnp.sum(xyz ** 2, axis=-1)[:, None, :]
          - 2.0 * jnp.einsum('bmd,bnd->bmn', centroids, xyz))
    _, group_idx = jax.lax.top_k(-d2, _K)
    idx_flat = (group_idx
                + (jnp.arange(_B, dtype=jnp.int32) * _N)[:, None, None]
                ).reshape(_B * _M * _K)
    xf, gx, gy, gz = _sc_gather(features.reshape(_B * _N, _FD), xyz, idx_flat)

    def _coord(g, c):
        return (g.reshape(b, _M, _K) - centroids[:, :, c:c + 1]).reshape(b * _M, _K)

    dx = _coord(gx, 0)
    dy = _coord(gy, 1)
    dz = _coord(gz, 2)
    out = _mlps(dx, dy, dz, xf, W1g, W2g, W1f, b1f, W2f, b2f)
    return out.reshape(b, _M, _GH + _FH)
